# baseline (speedup 1.0000x reference)
import functools

import jax
import jax.numpy as jnp
from jax import lax
from jax.experimental import pallas as pl
from jax.experimental.pallas import tpu as pltpu
from jax.experimental.pallas import tpu_sc as plsc

N_NODES = 10000
N_EDGES = 160000
NC = 2
NS = 16
NW = NC * NS
ROWS_PER_TILE = 632
PAD_NODES = ROWS_PER_TILE * NS
IDX_CHUNK = 20
N_CHUNKS = 10
NBUF = 4


def _agg_body(h_hbm, src_hbm, dst_hbm, zeros_hbm, out_hbm,
              src_v, dst_v, rows_v, acc,
              gsem0, gsem1, gsem2, gsem3, ssem0, ssem1, ssem2, ssem3,
              zsem, isem0, isem1, *, split_edges, edge_blk):
    cid = lax.axis_index("c")
    sid = lax.axis_index("s")

    zcp = pltpu.async_copy(
        zeros_hbm, acc.at[pl.ds(sid * ROWS_PER_TILE, ROWS_PER_TILE)], zsem)

    if split_edges:
        h_me = h_hbm
        widx = cid * NS + sid
    else:
        h_me = h_hbm.at[cid]
        widx = sid
    src_my = src_hbm.at[widx]
    dst_my = dst_hbm.at[widx]
    gsems = (gsem0, gsem1, gsem2, gsem3)
    ssems = (ssem0, ssem1, ssem2, ssem3)
    isems = (isem0, isem1)

    def idx_start(c, p):
        pltpu.async_copy(src_my.at[c], src_v.at[p], isems[p])
        pltpu.async_copy(dst_my.at[c], dst_v.at[p], isems[p])

    def idx_wait(c, p):
        pltpu.make_async_copy(src_my.at[c], src_v.at[p], isems[p]).wait()
        pltpu.make_async_copy(dst_my.at[c], dst_v.at[p], isems[p]).wait()

    def gather_start(p, j, b):
        pltpu.async_copy(h_me.at[src_v.at[p].at[j]], rows_v.at[b], gsems[b])

    def gather_wait(p, j, b):
        pltpu.make_async_copy(h_me.at[src_v.at[p].at[j]], rows_v.at[b],
                              gsems[b]).wait()

    def scat_start(p, j, b):
        pltpu.async_copy(rows_v.at[b], acc.at[dst_v.at[p].at[j]], ssems[b],
                         add=True)

    def scat_wait(p, j, b):
        pltpu.make_async_copy(rows_v.at[b], acc.at[dst_v.at[p].at[j]],
                              ssems[b]).wait()

    def step(c, p, j, b):
        gj = c * IDX_CHUNK + j

        @pl.when(gj >= 2)
        def _():
            p_prev = lax.select(j >= 2, p, 1 - p)
            j_prev = lax.select(j >= 2, j - 2, j - 2 + IDX_CHUNK)
            scat_wait(p_prev, j_prev, (b + 2) % NBUF)

        nxt = j + 2

        @pl.when(nxt < IDX_CHUNK)
        def _():
            gather_start(p, nxt, (b + 2) % NBUF)

        @pl.when((nxt >= IDX_CHUNK) & (c + 1 < N_CHUNKS))
        def _():
            @pl.when(nxt == IDX_CHUNK)
            def _():
                idx_wait(c + 1, 1 - p)

            gather_start(1 - p, nxt - IDX_CHUNK, (b + 2) % NBUF)

        gather_wait(p, j, b)
        scat_start(p, j, b)

    def one_chunk(c, p, load_pred):
        def quad(q, cr):
            step(c, p, 4 * q, 0)
            step(c, p, 4 * q + 1, 1)

            @pl.when((q == 0) & load_pred)
            def _():
                idx_start(c + 1, 1 - p)

            step(c, p, 4 * q + 2, 2)
            step(c, p, 4 * q + 3, 3)
            return cr

        lax.fori_loop(0, IDX_CHUNK // 4, quad, 0)

    def chunk_pair(qq, carry):
        c0 = 2 * qq
        one_chunk(c0, 0, c0 >= 1)
        one_chunk(c0 + 1, 1, c0 + 2 < N_CHUNKS)
        return carry

    idx_start(0, 0)
    idx_start(1, 1)
    zcp.wait()
    plsc.subcore_barrier()
    idx_wait(0, 0)
    gather_start(0, 0, 0)
    gather_start(0, 1, 1)

    lax.fori_loop(0, N_CHUNKS // 2, chunk_pair, 0)

    scat_wait(1, IDX_CHUNK - 2, 2)
    scat_wait(1, IDX_CHUNK - 1, 3)

    plsc.subcore_barrier()
    sl = pl.ds(sid * ROWS_PER_TILE, ROWS_PER_TILE)
    pltpu.sync_copy(acc.at[sl], out_hbm.at[cid].at[sl])


@functools.lru_cache(maxsize=None)
def _make_agg(split_edges: bool):
    edge_blk = 25 if split_edges else 50
    workers = NW if split_edges else NS
    mesh = plsc.VectorSubcoreMesh(core_axis_name="c", subcore_axis_name="s")
    return pl.kernel(
        functools.partial(_agg_body, split_edges=split_edges,
                          edge_blk=edge_blk),
        out_type=jax.ShapeDtypeStruct((NC, PAD_NODES, 128), jnp.float32),
        mesh=mesh,
        scratch_types=[
            pltpu.VMEM((2, IDX_CHUNK, edge_blk), jnp.int32),
            pltpu.VMEM((2, IDX_CHUNK, edge_blk), jnp.int32),
            pltpu.VMEM((NBUF, edge_blk, 128), jnp.float32),
            pltpu.VMEM_SHARED((PAD_NODES, 128), jnp.float32),
            pltpu.SemaphoreType.DMA,
            pltpu.SemaphoreType.DMA,
            pltpu.SemaphoreType.DMA,
            pltpu.SemaphoreType.DMA,
            pltpu.SemaphoreType.DMA,
            pltpu.SemaphoreType.DMA,
            pltpu.SemaphoreType.DMA,
            pltpu.SemaphoreType.DMA,
            pltpu.SemaphoreType.DMA,
            pltpu.SemaphoreType.DMA,
            pltpu.SemaphoreType.DMA,
        ],
        name=f"gcn_agg_{int(split_edges)}",
    )


def _sc_aggregate(h, src4d, dst4d, zeros_hbm):
    return _make_agg(h.ndim == 2)(h, src4d, dst4d, zeros_hbm)


ROW_BLK = 1000


def _layer_kern(x_ref, w_ref, b_ref, g_ref, bb_ref, o_ref, *, ln, split_out):
    if x_ref.ndim == 3:
        x = jnp.concatenate([x_ref[0], x_ref[1]], axis=-1)
    else:
        x = x_ref[...]
    if ln:
        mu = jnp.mean(x, axis=-1, keepdims=True)
        var = jnp.mean((x - mu) * (x - mu), axis=-1, keepdims=True)
        x = (x - mu) / jnp.sqrt(var + 1e-5) * g_ref[...] + bb_ref[...]
        x = jnp.maximum(x, 0.0)
    y = jnp.dot(x, w_ref[...], preferred_element_type=jnp.float32) + b_ref[...]
    if split_out:
        d_out = y.shape[1]
        o_ref[0] = y[:, : d_out // 2]
        o_ref[1] = y[:, d_out // 2:]
    else:
        o_ref[...] = y


def _tc_layer(x_split, W, b, g, bb, *, ln, split_out=True):
    if x_split.ndim == 3:
        d_in = 2 * x_split.shape[2]
        in_spec = pl.BlockSpec((2, ROW_BLK, d_in // 2), lambda i: (0, i, 0))
    else:
        d_in = x_split.shape[1]
        in_spec = pl.BlockSpec((ROW_BLK, d_in), lambda i: (i, 0))
    d_out = W.shape[1]
    grid = (N_NODES // ROW_BLK,)
    if split_out:
        out_spec = pl.BlockSpec((2, ROW_BLK, d_out // 2), lambda i: (0, i, 0))
        out_shape = jax.ShapeDtypeStruct((2, N_NODES, d_out // 2), jnp.float32)
    else:
        out_spec = pl.BlockSpec((ROW_BLK, d_out), lambda i: (i, 0))
        out_shape = jax.ShapeDtypeStruct((N_NODES, d_out), jnp.float32)
    return pl.pallas_call(
        functools.partial(_layer_kern, ln=ln, split_out=split_out),
        grid=grid,
        in_specs=[
            in_spec,
            pl.BlockSpec((d_in, d_out), lambda i: (0, 0)),
            pl.BlockSpec((d_out,), lambda i: (0,)),
            pl.BlockSpec((d_in,), lambda i: (0,)),
            pl.BlockSpec((d_in,), lambda i: (0,)),
        ],
        out_specs=out_spec,
        out_shape=out_shape,
    )(x_split, W, b, g, bb)


def _lsm_kern(a_ref, o_ref, *, n_class):
    x = a_ref[0] + a_ref[1]
    col = lax.broadcasted_iota(jnp.int32, x.shape, 1)
    valid = col < n_class
    xm = jnp.where(valid, x, -jnp.inf)
    m = jnp.max(xm, axis=-1, keepdims=True)
    ex = jnp.where(valid, jnp.exp(x - m), 0.0)
    lse = jnp.log(jnp.sum(ex, axis=-1, keepdims=True)) + m
    o_ref[...] = (x - lse)[:, :n_class]


def _tc_log_softmax(a_split, n_class):
    h = a_split.shape[2]
    grid = (N_NODES // ROW_BLK,)
    return pl.pallas_call(
        functools.partial(_lsm_kern, n_class=n_class),
        grid=grid,
        in_specs=[pl.BlockSpec((2, ROW_BLK, h), lambda i: (0, i, 0))],
        out_specs=pl.BlockSpec((ROW_BLK, n_class), lambda i: (i, 0)),
        out_shape=jax.ShapeDtypeStruct((N_NODES, n_class), jnp.float32),
    )(a_split)


def kernel(feats, edge_index, W1, b1, W2, b2, W3, b3, W4, b4, ln_g, ln_b):
    n, d_in = feats.shape
    n_class = W4.shape[1]

    src = edge_index[0].astype(jnp.int32)
    dst = edge_index[1].astype(jnp.int32)
    src_col = src.reshape(NS, N_CHUNKS, IDX_CHUNK, 50)
    dst_col = dst.reshape(NS, N_CHUNKS, IDX_CHUNK, 50)
    src_edge = src.reshape(NW, N_CHUNKS, IDX_CHUNK, 25)
    dst_edge = dst.reshape(NW, N_CHUNKS, IDX_CHUNK, 25)

    W4p = jnp.pad(W4, ((0, 0), (0, 128 - n_class)))
    b4p = jnp.pad(b4, (0, 128 - n_class))

    zeros128 = jnp.zeros((ROWS_PER_TILE, 128), jnp.float32)

    h1 = _tc_layer(feats, W1, b1, ln_g, ln_b, ln=False)
    a1 = _sc_aggregate(h1, src_col, dst_col, zeros128)
    h2 = _tc_layer(a1, W2, b2, ln_g, ln_b, ln=True)
    a2 = _sc_aggregate(h2, src_col, dst_col, zeros128)
    h3 = _tc_layer(a2, W3, b3, ln_g, ln_b, ln=True)
    a3 = _sc_aggregate(h3, src_col, dst_col, zeros128)
    h4 = _tc_layer(a3, W4p, b4p, ln_g, ln_b, ln=True, split_out=False)
    a4 = _sc_aggregate(h4, src_edge, dst_edge, zeros128)
    return _tc_log_softmax(a4, n_class)


# speedup vs baseline: 1.0163x; 1.0163x over previous
import functools

import jax
import jax.numpy as jnp
from jax import lax
from jax.experimental import pallas as pl
from jax.experimental.pallas import tpu as pltpu
from jax.experimental.pallas import tpu_sc as plsc

N_NODES = 10000
N_EDGES = 160000
NC = 2
NS = 16
EDGE_BLK = 125
EDGES_PER_TILE = N_EDGES // NS
BLKS_PER_TILE = EDGES_PER_TILE // EDGE_BLK
ROWS_PER_TILE = 632
PAD_NODES = ROWS_PER_TILE * NS


def _agg_body(h_hbm, src_hbm, dst_hbm, zeros_hbm, out_hbm,
              src_v, dst_v, rows_v, acc, gsem0, gsem1, ssem0, ssem1, zsem,
              isem0, isem1, *, split_edges, idx_chunk):
    cid = lax.axis_index("c")
    sid = lax.axis_index("s")

    zcp = pltpu.async_copy(
        zeros_hbm, acc.at[pl.ds(sid * ROWS_PER_TILE, ROWS_PER_TILE)], zsem)

    if split_edges:
        h_me = h_hbm
        blks_per_tile = BLKS_PER_TILE // 2
        widx = cid * NS + sid
    else:
        h_me = h_hbm.at[cid]
        blks_per_tile = BLKS_PER_TILE
        widx = sid
    n_chunks = blks_per_tile // idx_chunk
    src_my = src_hbm.at[widx]
    dst_my = dst_hbm.at[widx]
    gsems = (gsem0, gsem1)
    ssems = (ssem0, ssem1)
    isems = (isem0, isem1)

    def idx_start(c, p):
        pltpu.async_copy(src_my.at[c], src_v.at[p], isems[p])
        pltpu.async_copy(dst_my.at[c], dst_v.at[p], isems[p])

    def idx_wait(c, p):
        pltpu.make_async_copy(src_my.at[c], src_v.at[p], isems[p]).wait()
        pltpu.make_async_copy(dst_my.at[c], dst_v.at[p], isems[p]).wait()

    def gather_start(p, j, b):
        pltpu.async_copy(h_me.at[src_v.at[p].at[j]], rows_v.at[b], gsems[b])

    def gather_wait(p, j, b):
        pltpu.make_async_copy(h_me.at[src_v.at[p].at[j]], rows_v.at[b],
                              gsems[b]).wait()

    def scat_start(p, j, b):
        pltpu.async_copy(rows_v.at[b], acc.at[dst_v.at[p].at[j]], ssems[b],
                         add=True)

    def scat_wait(p, j, b):
        pltpu.make_async_copy(rows_v.at[b], acc.at[dst_v.at[p].at[j]],
                              ssems[b]).wait()

    def step(c, p, j, b):
        gj = c * idx_chunk + j

        @pl.when(gj >= 1)
        def _():
            p_prev = lax.select(j >= 1, p, 1 - p)
            j_prev = lax.select(j >= 1, j - 1, idx_chunk - 1)
            scat_wait(p_prev, j_prev, 1 - b)

        nxt = j + 1

        @pl.when(nxt < idx_chunk)
        def _():
            gather_start(p, nxt, 1 - b)

        @pl.when((nxt == idx_chunk) & (c + 1 < n_chunks))
        def _():
            idx_wait(c + 1, 1 - p)
            gather_start(1 - p, 0, 1 - b)

        gather_wait(p, j, b)
        scat_start(p, j, b)

    def one_chunk(c, p, load_pred):
        def pair(g, cr):
            step(c, p, 2 * g, 0)

            @pl.when((g == 0) & load_pred)
            def _():
                idx_start(c + 1, 1 - p)

            step(c, p, 2 * g + 1, 1)
            return cr

        lax.fori_loop(0, idx_chunk // 2, pair, 0)

    def chunk_pair(q, carry):
        c0 = 2 * q
        one_chunk(c0, 0, c0 >= 1)
        one_chunk(c0 + 1, 1, c0 + 2 < n_chunks)
        return carry

    idx_start(0, 0)
    idx_start(1, 1)
    zcp.wait()
    plsc.subcore_barrier()
    idx_wait(0, 0)
    gather_start(0, 0, 0)

    lax.fori_loop(0, n_chunks // 2, chunk_pair, 0)

    scat_wait(1, idx_chunk - 1, 1)

    plsc.subcore_barrier()
    sl = pl.ds(sid * ROWS_PER_TILE, ROWS_PER_TILE)
    pltpu.sync_copy(acc.at[sl], out_hbm.at[cid].at[sl])


@functools.lru_cache(maxsize=None)
def _make_agg(split_edges: bool):
    idx_chunk = 20
    mesh = plsc.VectorSubcoreMesh(core_axis_name="c", subcore_axis_name="s")
    return pl.kernel(
        functools.partial(_agg_body, split_edges=split_edges,
                          idx_chunk=idx_chunk),
        out_type=jax.ShapeDtypeStruct((NC, PAD_NODES, 128), jnp.float32),
        mesh=mesh,
        scratch_types=[
            pltpu.VMEM((2, idx_chunk, EDGE_BLK), jnp.int32),
            pltpu.VMEM((2, idx_chunk, EDGE_BLK), jnp.int32),
            pltpu.VMEM((2, EDGE_BLK, 128), jnp.float32),
            pltpu.VMEM_SHARED((PAD_NODES, 128), jnp.float32),
            pltpu.SemaphoreType.DMA,
            pltpu.SemaphoreType.DMA,
            pltpu.SemaphoreType.DMA,
            pltpu.SemaphoreType.DMA,
            pltpu.SemaphoreType.DMA,
            pltpu.SemaphoreType.DMA,
            pltpu.SemaphoreType.DMA,
        ],
        name=f"gcn_agg_{int(split_edges)}",
    )


def _sc_aggregate(h, src4d, dst4d, zeros_hbm):
    return _make_agg(h.ndim == 2)(h, src4d, dst4d, zeros_hbm)


ROW_BLK = 1000


def _layer_kern(x_ref, w_ref, b_ref, g_ref, bb_ref, o_ref, *, ln, split_out):
    if x_ref.ndim == 3:
        x = jnp.concatenate([x_ref[0], x_ref[1]], axis=-1)
    else:
        x = x_ref[...]
    if ln:
        mu = jnp.mean(x, axis=-1, keepdims=True)
        var = jnp.mean((x - mu) * (x - mu), axis=-1, keepdims=True)
        x = (x - mu) / jnp.sqrt(var + 1e-5) * g_ref[...] + bb_ref[...]
        x = jnp.maximum(x, 0.0)
    y = jnp.dot(x, w_ref[...], preferred_element_type=jnp.float32) + b_ref[...]
    if split_out:
        d_out = y.shape[1]
        o_ref[0] = y[:, : d_out // 2]
        o_ref[1] = y[:, d_out // 2:]
    else:
        o_ref[...] = y


def _tc_layer(x_split, W, b, g, bb, *, ln, split_out=True):
    if x_split.ndim == 3:
        d_in = 2 * x_split.shape[2]
        in_spec = pl.BlockSpec((2, ROW_BLK, d_in // 2), lambda i: (0, i, 0))
    else:
        d_in = x_split.shape[1]
        in_spec = pl.BlockSpec((ROW_BLK, d_in), lambda i: (i, 0))
    d_out = W.shape[1]
    grid = (N_NODES // ROW_BLK,)
    if split_out:
        out_spec = pl.BlockSpec((2, ROW_BLK, d_out // 2), lambda i: (0, i, 0))
        out_shape = jax.ShapeDtypeStruct((2, N_NODES, d_out // 2), jnp.float32)
    else:
        out_spec = pl.BlockSpec((ROW_BLK, d_out), lambda i: (i, 0))
        out_shape = jax.ShapeDtypeStruct((N_NODES, d_out), jnp.float32)
    return pl.pallas_call(
        functools.partial(_layer_kern, ln=ln, split_out=split_out),
        grid=grid,
        in_specs=[
            in_spec,
            pl.BlockSpec((d_in, d_out), lambda i: (0, 0)),
            pl.BlockSpec((d_out,), lambda i: (0,)),
            pl.BlockSpec((d_in,), lambda i: (0,)),
            pl.BlockSpec((d_in,), lambda i: (0,)),
        ],
        out_specs=out_spec,
        out_shape=out_shape,
    )(x_split, W, b, g, bb)


def _lsm_kern(a_ref, o_ref, *, n_class):
    x = a_ref[0] + a_ref[1]
    col = lax.broadcasted_iota(jnp.int32, x.shape, 1)
    valid = col < n_class
    xm = jnp.where(valid, x, -jnp.inf)
    m = jnp.max(xm, axis=-1, keepdims=True)
    ex = jnp.where(valid, jnp.exp(x - m), 0.0)
    lse = jnp.log(jnp.sum(ex, axis=-1, keepdims=True)) + m
    o_ref[...] = (x - lse)[:, :n_class]


def _tc_log_softmax(a_split, n_class):
    h = a_split.shape[2]
    grid = (N_NODES // ROW_BLK,)
    return pl.pallas_call(
        functools.partial(_lsm_kern, n_class=n_class),
        grid=grid,
        in_specs=[pl.BlockSpec((2, ROW_BLK, h), lambda i: (0, i, 0))],
        out_specs=pl.BlockSpec((ROW_BLK, n_class), lambda i: (i, 0)),
        out_shape=jax.ShapeDtypeStruct((N_NODES, n_class), jnp.float32),
    )(a_split)


def kernel(feats, edge_index, W1, b1, W2, b2, W3, b3, W4, b4, ln_g, ln_b):
    n, d_in = feats.shape
    n_class = W4.shape[1]

    src = edge_index[0].astype(jnp.int32)
    dst = edge_index[1].astype(jnp.int32)
    src_col = src.reshape(NS, 4, 20, EDGE_BLK)
    dst_col = dst.reshape(NS, 4, 20, EDGE_BLK)
    src_edge = src.reshape(NC * NS, 2, 20, EDGE_BLK)
    dst_edge = dst.reshape(NC * NS, 2, 20, EDGE_BLK)

    W4p = jnp.pad(W4, ((0, 0), (0, 128 - n_class)))
    b4p = jnp.pad(b4, (0, 128 - n_class))

    zeros128 = jnp.zeros((ROWS_PER_TILE, 128), jnp.float32)

    h1 = _tc_layer(feats, W1, b1, ln_g, ln_b, ln=False)
    a1 = _sc_aggregate(h1, src_col, dst_col, zeros128)
    h2 = _tc_layer(a1, W2, b2, ln_g, ln_b, ln=True)
    a2 = _sc_aggregate(h2, src_col, dst_col, zeros128)
    h3 = _tc_layer(a2, W3, b3, ln_g, ln_b, ln=True)
    a3 = _sc_aggregate(h3, src_col, dst_col, zeros128)
    h4 = _tc_layer(a3, W4p, b4p, ln_g, ln_b, ln=True, split_out=False)
    a4 = _sc_aggregate(h4, src_edge, dst_edge, zeros128)
    return _tc_log_softmax(a4, n_class)
